# baseline (speedup 1.0000x reference)
import functools

import jax
import jax.numpy as jnp
from jax import lax
from jax.experimental import pallas as pl
from jax.experimental.pallas import tpu as pltpu
from jax.experimental.pallas import tpu_sc as plsc

B = 16384
D = 512
L = 16
NC = 2
NS = 16
NW = NC * NS
BPW = B // NW
CHUNK = 16
NCHUNKS = BPW // CHUNK
NBUF = 4

_mesh = plsc.VectorSubcoreMesh(core_axis_name="c", subcore_axis_name="s")


@functools.partial(
    pl.kernel,
    mesh=_mesh,
    out_type=jax.ShapeDtypeStruct((NW, L), jnp.float32),
    scratch_types=[
        pltpu.VMEM((NBUF, CHUNK), jnp.int32),
        pltpu.VMEM((NBUF, CHUNK, D), jnp.float32),
        pltpu.VMEM((NBUF, CHUNK, D), jnp.float32),
        pltpu.VMEM((L,), jnp.float32),
    ] + [pltpu.SemaphoreType.DMA] * (2 * NBUF),
)
def _center_loss_partials(feat_hbm, lab_hbm, cent_hbm, out_hbm,
                          idx_v, rows_v, feat_v, acc_v, *sems):
    gsem = sems[:NBUF]
    fsem = sems[NBUF:]
    wid = lax.axis_index("s") * NC + lax.axis_index("c")
    base = wid * BPW

    def issue(c, b):
        off = base + c * CHUNK
        pltpu.sync_copy(lab_hbm.at[pl.ds(off, CHUNK)], idx_v.at[b])
        pltpu.async_copy(cent_hbm.at[idx_v.at[b]], rows_v.at[b], gsem[b])
        pltpu.async_copy(feat_hbm.at[pl.ds(off, CHUNK)], feat_v.at[b], fsem[b])

    def drain(c, b):
        off = base + c * CHUNK
        pltpu.make_async_copy(cent_hbm.at[idx_v.at[b]], rows_v.at[b], gsem[b]).wait()
        pltpu.make_async_copy(feat_hbm.at[pl.ds(off, CHUNK)], feat_v.at[b], fsem[b]).wait()

    for b in range(NBUF):
        issue(b, b)

    def outer(g, acc4):
        c0 = g * NBUF
        for b in range(NBUF):
            drain(c0 + b, b)

            def row_body(r, a4):
                a = list(a4)
                for j in range(D // L):
                    fv = feat_v[b, r, pl.ds(j * L, L)]
                    cv = rows_v[b, r, pl.ds(j * L, L)]
                    d = fv - cv
                    a[j % 4] = a[j % 4] + d * d
                return tuple(a)

            acc4 = lax.fori_loop(0, CHUNK, row_body, acc4)

            nxt = c0 + b + NBUF

            @pl.when(nxt < NCHUNKS)
            def _():
                issue(nxt, b)

        return acc4

    z = jnp.zeros((L,), jnp.float32)
    acc4 = lax.fori_loop(0, NCHUNKS // NBUF, outer, (z, z, z, z))
    acc_v[...] = acc4[0] + acc4[1] + acc4[2] + acc4[3]
    pltpu.sync_copy(acc_v, out_hbm.at[wid])


def kernel(features, labels, centers):
    partials = _center_loss_partials(features, labels.astype(jnp.int32), centers)
    return jnp.sum(partials) * (1.0 / B)


# speedup vs baseline: 1.2018x; 1.2018x over previous
import functools

import jax
import jax.numpy as jnp
from jax import lax
from jax.experimental import pallas as pl
from jax.experimental.pallas import tpu as pltpu
from jax.experimental.pallas import tpu_sc as plsc

B = 16384
D = 512
L = 16
NC = 2
NS = 16
NW = NC * NS
BPW = B // NW
CHUNK = 32
NCHUNKS = BPW // CHUNK
NBUF = 3

_mesh = plsc.VectorSubcoreMesh(core_axis_name="c", subcore_axis_name="s")


@functools.partial(
    pl.kernel,
    mesh=_mesh,
    out_type=jax.ShapeDtypeStruct((NW, L), jnp.float32),
    scratch_types=[
        pltpu.VMEM((BPW,), jnp.int32),
        pltpu.VMEM((NBUF, CHUNK, D), jnp.float32),
        pltpu.VMEM((NBUF, CHUNK, D), jnp.float32),
        pltpu.VMEM((L,), jnp.float32),
    ] + [pltpu.SemaphoreType.DMA] * (2 * NBUF),
)
def _center_loss_partials(feat_hbm, lab_hbm, cent_hbm, out_hbm,
                          idx_v, rows_v, feat_v, acc_v, *sems):
    gsem = sems[:NBUF]
    fsem = sems[NBUF:]
    wid = lax.axis_index("s") * NC + lax.axis_index("c")
    base = wid * BPW
    pltpu.sync_copy(lab_hbm.at[pl.ds(base, BPW)], idx_v)

    def issue(c, b):
        off = base + c * CHUNK
        pltpu.async_copy(cent_hbm.at[idx_v.at[pl.ds(c * CHUNK, CHUNK)]],
                         rows_v.at[b], gsem[b])
        pltpu.async_copy(feat_hbm.at[pl.ds(off, CHUNK)], feat_v.at[b], fsem[b])

    def drain(c, b):
        off = base + c * CHUNK
        pltpu.make_async_copy(cent_hbm.at[idx_v.at[pl.ds(c * CHUNK, CHUNK)]],
                              rows_v.at[b], gsem[b]).wait()
        pltpu.make_async_copy(feat_hbm.at[pl.ds(off, CHUNK)], feat_v.at[b],
                              fsem[b]).wait()

    for b in range(NBUF):
        issue(b, b)

    def outer(g, acc4):
        c0 = g * NBUF
        for b in range(NBUF):
            drain(c0 + b, b)

            def row_body(r, a4):
                a = list(a4)
                for j in range(D // L):
                    fv = feat_v[b, r, pl.ds(j * L, L)]
                    cv = rows_v[b, r, pl.ds(j * L, L)]
                    d = fv - cv
                    a[j % 4] = a[j % 4] + d * d
                return tuple(a)

            acc4 = lax.fori_loop(0, CHUNK, row_body, acc4)

            nxt = c0 + b + NBUF

            @pl.when(nxt < NCHUNKS)
            def _():
                issue(nxt, b)

        return acc4

    z = jnp.zeros((L,), jnp.float32)
    acc4 = lax.fori_loop(0, NCHUNKS // NBUF, outer, (z, z, z, z))
    for c in range((NCHUNKS // NBUF) * NBUF, NCHUNKS):
        b = c % NBUF
        drain(c, b)

        def row_body_t(r, a4):
            a = list(a4)
            for j in range(D // L):
                fv = feat_v[b, r, pl.ds(j * L, L)]
                cv = rows_v[b, r, pl.ds(j * L, L)]
                d = fv - cv
                a[j % 4] = a[j % 4] + d * d
            return tuple(a)

        acc4 = lax.fori_loop(0, CHUNK, row_body_t, acc4)

    acc_v[...] = acc4[0] + acc4[1] + acc4[2] + acc4[3]
    pltpu.sync_copy(acc_v, out_hbm.at[wid])


def kernel(features, labels, centers):
    partials = _center_loss_partials(features, labels.astype(jnp.int32), centers)
    return jnp.sum(partials) * (1.0 / B)
